# baseline (speedup 1.0000x reference)
import functools

import jax
import jax.numpy as jnp
from jax import lax
from jax.experimental import pallas as pl
from jax.experimental.pallas import tpu as pltpu
from jax.experimental.pallas import tpu_sc as plsc

_F = 256
_D = 64
_PAIRS_PER_W = 8


def _sc_body(Bs, HWD, row_hbm, col_hbm, dep_hbm, out_hbm,
             buf_a, buf_b, rbuf, cbuf8, spmem, sem_a, sem_b, sem_c, sem_d):
    cid = lax.axis_index("c")
    sid = lax.axis_index("s")
    wid = sid * 2 + cid
    q0 = wid * _PAIRS_PER_W
    i = q0 // 16
    j0 = lax.rem(q0, 16)

    bufs = (buf_a, buf_b)
    stream_sems = (sem_a, sem_b)
    local_sems = (sem_c, sem_d)

    pltpu.sync_copy(row_hbm.at[i], rbuf)
    pltpu.sync_copy(col_hbm.at[pl.ds(j0, _PAIRS_PER_W)], cbuf8)

    rv = [rbuf[pl.ds(c * 16, 16)] for c in range(_F // 16)]
    for buf in bufs:
        pltpu.sync_copy(dep_hbm, buf.at[:, pl.ds(2 * _F, _F)])

        def fill_rows(r, carry):
            for c in range(_F // 16):
                buf[r, pl.ds(c * 16, 16)] = rv[c]
            return carry

        lax.fori_loop(0, _D, fill_rows, 0)

    stream_pending = [[], []]
    local_pending = [[], []]
    for t in range(_PAIRS_PER_W):
        slot = t % 2
        buf = bufs[slot]
        for dsc in stream_pending[slot]:
            dsc.wait()
        stream_pending[slot] = []

        cv = [cbuf8[t, pl.ds(c * 16, 16)] for c in range(_F // 16)]

        def fill_col(r, carry):
            for c in range(_F // 16):
                buf[r, pl.ds(_F + c * 16, 16)] = cv[c]
            return carry

        lax.fori_loop(0, _D, fill_col, 0)

        base = i * 1024 + (j0 + t) * _D
        if t % 2 == 0 or t == _PAIRS_PER_W - 1:
            for b in range(Bs):
                dst = out_hbm.at[pl.ds(base + b * HWD, _D)]
                stream_pending[slot].append(
                    pltpu.async_copy(buf, dst, stream_sems[slot]))
        else:
            for qtr in range(4):
                pslot = qtr % 2
                for dsc in local_pending[pslot]:
                    dsc.wait()
                local_pending[pslot] = []
                pltpu.sync_copy(buf.at[pl.ds(qtr * 16, 16)], spmem.at[sid, pslot])
                for b in range(Bs):
                    dst = out_hbm.at[pl.ds(base + qtr * 16 + b * HWD, 16)]
                    local_pending[pslot].append(
                        pltpu.async_copy(spmem.at[sid, pslot], dst,
                                         local_sems[pslot]))

    for lst in (stream_pending[0], stream_pending[1],
                local_pending[0], local_pending[1]):
        for dsc in lst:
            dsc.wait()


def kernel(B, h, w, d, x, row_embed, col_embed, dep_embed):
    H, F = row_embed.shape
    W = col_embed.shape[0]
    D = dep_embed.shape[0]
    Bs = x.shape[0]
    HWD = H * W * D
    mesh = plsc.VectorSubcoreMesh(core_axis_name="c", subcore_axis_name="s")
    sc_call = functools.partial(
        pl.kernel,
        mesh=mesh,
        out_type=jax.ShapeDtypeStruct((Bs * HWD, 3 * F), jnp.float32),
        scratch_types=[
            pltpu.VMEM((D, 3 * F), jnp.float32),
            pltpu.VMEM((D, 3 * F), jnp.float32),
            pltpu.VMEM((F,), jnp.float32),
            pltpu.VMEM((_PAIRS_PER_W, F), jnp.float32),
            pltpu.VMEM_SHARED((16, 2, 16, 3 * F), jnp.float32),
            pltpu.SemaphoreType.DMA,
            pltpu.SemaphoreType.DMA,
            pltpu.SemaphoreType.DMA,
            pltpu.SemaphoreType.DMA,
        ],
    )(functools.partial(_sc_body, Bs, HWD))
    out = sc_call(row_embed, col_embed, dep_embed)
    return out.reshape(Bs, HWD, 3 * F)


# speedup vs baseline: 1.0046x; 1.0046x over previous
import functools

import jax
import jax.numpy as jnp
from jax import lax
from jax.experimental import pallas as pl
from jax.experimental.pallas import tpu as pltpu
from jax.experimental.pallas import tpu_sc as plsc

_F = 256
_D = 64
_PAIRS_PER_W = 8


def _sc_body(Bs, HWD, row_hbm, col_hbm, dep_hbm, out_hbm,
             buf_a, buf_b, rbuf, cbuf8, spmem, sem_a, sem_b, sem_c, sem_d):
    cid = lax.axis_index("c")
    sid = lax.axis_index("s")
    wid = sid * 2 + cid
    q0 = wid * _PAIRS_PER_W
    i = q0 // 16
    j0 = lax.rem(q0, 16)

    bufs = (buf_a, buf_b)
    stream_sems = (sem_a, sem_b)
    local_sems = (sem_c, sem_d)

    pltpu.sync_copy(row_hbm.at[i], rbuf)
    pltpu.sync_copy(col_hbm.at[pl.ds(j0, _PAIRS_PER_W)], cbuf8)

    rv = [rbuf[pl.ds(c * 16, 16)] for c in range(_F // 16)]
    for buf in bufs:
        pltpu.sync_copy(dep_hbm, buf.at[:, pl.ds(2 * _F, _F)])

        def fill_rows(r, carry):
            for c in range(_F // 16):
                buf[r, pl.ds(c * 16, 16)] = rv[c]
            return carry

        lax.fori_loop(0, _D, fill_rows, 0)

    stream_pending = [[], []]
    local_pending = [[], []]
    for t in range(_PAIRS_PER_W):
        slot = t % 2
        buf = bufs[slot]
        for dsc in stream_pending[slot]:
            dsc.wait()
        stream_pending[slot] = []

        cv = [cbuf8[t, pl.ds(c * 16, 16)] for c in range(_F // 16)]

        def fill_col(r, carry):
            for c in range(_F // 16):
                buf[r, pl.ds(_F + c * 16, 16)] = cv[c]
            return carry

        lax.fori_loop(0, _D, fill_col, 0)

        base = i * 1024 + (j0 + t) * _D
        if t % 2 == 0:
            for b in range(Bs):
                dst = out_hbm.at[pl.ds(base + b * HWD, _D)]
                stream_pending[slot].append(
                    pltpu.async_copy(buf, dst, stream_sems[slot]))
        else:
            for qtr in range(4):
                pslot = qtr % 2
                for dsc in local_pending[pslot]:
                    dsc.wait()
                local_pending[pslot] = []
                pltpu.sync_copy(buf.at[pl.ds(qtr * 16, 16)], spmem.at[sid, pslot])
                for b in range(Bs):
                    dst = out_hbm.at[pl.ds(base + qtr * 16 + b * HWD, 16)]
                    local_pending[pslot].append(
                        pltpu.async_copy(spmem.at[sid, pslot], dst,
                                         local_sems[pslot]))

    for lst in (stream_pending[0], stream_pending[1],
                local_pending[0], local_pending[1]):
        for dsc in lst:
            dsc.wait()


def kernel(B, h, w, d, x, row_embed, col_embed, dep_embed):
    H, F = row_embed.shape
    W = col_embed.shape[0]
    D = dep_embed.shape[0]
    Bs = x.shape[0]
    HWD = H * W * D
    mesh = plsc.VectorSubcoreMesh(core_axis_name="c", subcore_axis_name="s")
    sc_call = functools.partial(
        pl.kernel,
        mesh=mesh,
        out_type=jax.ShapeDtypeStruct((Bs * HWD, 3 * F), jnp.float32),
        scratch_types=[
            pltpu.VMEM((D, 3 * F), jnp.float32),
            pltpu.VMEM((D, 3 * F), jnp.float32),
            pltpu.VMEM((F,), jnp.float32),
            pltpu.VMEM((_PAIRS_PER_W, F), jnp.float32),
            pltpu.VMEM_SHARED((16, 2, 16, 3 * F), jnp.float32),
            pltpu.SemaphoreType.DMA,
            pltpu.SemaphoreType.DMA,
            pltpu.SemaphoreType.DMA,
            pltpu.SemaphoreType.DMA,
        ],
    )(functools.partial(_sc_body, Bs, HWD))
    out = sc_call(row_embed, col_embed, dep_embed)
    return out.reshape(Bs, HWD, 3 * F)
